# Initial kernel scaffold; baseline (speedup 1.0000x reference)
import jax
import jax.numpy as jnp
from jax.experimental import pallas as pl


def kernel(x, edge_index, batch, W1, att_src1, att_dst1, b1, W2, att_src2, att_dst2, b2, W3, att_src3, att_dst3, b3, W4, att_src4, att_dst4, b4, lin1_W, lin1_b, lin2_W, lin2_b):
    raise NotImplementedError("write your pallas kernel here")


# speedup vs baseline: 15.6383x; 15.6383x over previous
import functools

import jax
import jax.numpy as jnp
from jax import lax
from jax.experimental import pallas as pl
from jax.experimental.pallas import tpu as pltpu
from jax.experimental.pallas import tpu_sc as plsc

N = 10000
NP = 10240
E = 320000
EP = 327680
BATCHES = 128
SUB = 2048
GRP = 128
RB = 1024
STRIPE = NP // 16


def _elu(v):
    return jnp.where(v > 0, v, jnp.exp(jnp.minimum(v, 0.0)) - 1.0)


def _scores(h, asrc_ref, adst_ref):
    ss = jnp.sum(h * asrc_ref[...], axis=1)
    sd = jnp.sum(h * adst_ref[...], axis=1)
    al = ss + sd
    exself = jnp.exp(jnp.where(al >= 0, al, 0.2 * al))
    return ss, sd, exself


def _pre1_body(x_ref, W_ref, asrc_ref, adst_ref,
               h_ref, ssrc_ref, sdst_ref, exself_ref):
    h = jnp.dot(x_ref[...], W_ref[...], preferred_element_type=jnp.float32)
    h_ref[...] = h
    ss, sd, exself = _scores(h, asrc_ref, adst_ref)
    ssrc_ref[...] = ss
    sdst_ref[...] = sd
    exself_ref[...] = exself


def _pre_sum_body(a0_ref, a1_ref, hprev_ref, exs_ref, den_ref, b_ref,
                  W_ref, asrc_ref, adst_ref,
                  h_ref, ssrc_ref, sdst_ref, exself_ref):
    cs = (exs_ref[...] / (den_ref[...] + 1e-16)).reshape(-1, 1)
    x = _elu(a0_ref[...] + a1_ref[...] + hprev_ref[...] * cs + b_ref[...])
    h = jnp.dot(x, W_ref[...], preferred_element_type=jnp.float32)
    h_ref[...] = h
    ss, sd, exself = _scores(h, asrc_ref, adst_ref)
    ssrc_ref[...] = ss
    sdst_ref[...] = sd
    exself_ref[...] = exself


def _pre4_body(a0_ref, a1_ref, hprev_ref, exs_ref, den_ref, b_ref,
               W_ref, asrc_ref, adst_ref,
               h0_ref, h1_ref, ssrc_ref, sdst_ref, exself_ref):
    cs = (exs_ref[...] / (den_ref[...] + 1e-16)).reshape(-1, 1)
    x = _elu(a0_ref[...] + a1_ref[...] + hprev_ref[...] * cs + b_ref[...])
    h = jnp.dot(x, W_ref[...], preferred_element_type=jnp.float32)
    h0_ref[...] = h[:, :128]
    h1_ref[...] = h[:, 128:]
    ss, sd, exself = _scores(h, asrc_ref, adst_ref)
    ssrc_ref[...] = ss
    sdst_ref[...] = sd
    exself_ref[...] = exself


def _head_body(a0_ref, a1_ref, h0_ref, h1_ref, exs_ref, den_ref, b_ref,
               batch_ref, l1W_ref, l1b_ref, l2W_ref, l2b_ref,
               out_ref, sums_ref, counts_ref):
    i = pl.program_id(0)

    @pl.when(i == 0)
    def _():
        sums_ref[...] = jnp.zeros_like(sums_ref)
        counts_ref[...] = jnp.zeros_like(counts_ref)

    cs = (exs_ref[...] / (den_ref[...] + 1e-16)).reshape(-1, 1)
    agg = jnp.concatenate([a0_ref[...], a1_ref[...]], axis=1)
    hprev = jnp.concatenate([h0_ref[...], h1_ref[...]], axis=1)
    x = _elu(agg + hprev * cs + b_ref[...])
    bb = batch_ref[...].reshape(RB, 1)
    P = (bb == lax.broadcasted_iota(jnp.int32, (RB, BATCHES), 1)
         ).astype(jnp.float32)
    sums_ref[...] += lax.dot_general(
        P, x, (((0,), (0,)), ((), ())), preferred_element_type=jnp.float32)
    counts_ref[...] += jnp.sum(P, axis=0).reshape(1, BATCHES)

    @pl.when(i == pl.num_programs(0) - 1)
    def _():
        counts = jnp.maximum(counts_ref[...].reshape(BATCHES, 1), 1.0)
        pooled = sums_ref[...] / counts
        hh = _elu(jnp.dot(pooled, l1W_ref[...],
                          preferred_element_type=jnp.float32) + l1b_ref[...])
        out_ref[...] = jnp.dot(hh, l2W_ref[...],
                               preferred_element_type=jnp.float32) + l2b_ref[...]


def _row_spec(width):
    return pl.BlockSpec((RB, width), lambda i: (i, 0))


def _vec_spec():
    return pl.BlockSpec((RB,), lambda i: (i,))


def _full_spec(shape):
    return pl.BlockSpec(shape, lambda i: tuple(0 for _ in shape))


def _tc_pre1(xp, W, asrc, adst, dout):
    return pl.pallas_call(
        _pre1_body,
        grid=(NP // RB,),
        in_specs=[_row_spec(128), _full_spec(W.shape),
                  _full_spec((1, dout)), _full_spec((1, dout))],
        out_specs=[_row_spec(dout), _vec_spec(), _vec_spec(), _vec_spec()],
        out_shape=[jax.ShapeDtypeStruct((NP, dout), jnp.float32),
                   jax.ShapeDtypeStruct((NP,), jnp.float32),
                   jax.ShapeDtypeStruct((NP,), jnp.float32),
                   jax.ShapeDtypeStruct((NP,), jnp.float32)],
    )(xp, W, asrc.reshape(1, -1), adst.reshape(1, -1))


def _tc_pre_sum(a0, a1, hprev, exs, den, b, W, asrc, adst, din, dout):
    return pl.pallas_call(
        _pre_sum_body,
        grid=(NP // RB,),
        in_specs=[_row_spec(din), _row_spec(din), _row_spec(din),
                  _vec_spec(), _vec_spec(), _full_spec((1, din)),
                  _full_spec(W.shape),
                  _full_spec((1, dout)), _full_spec((1, dout))],
        out_specs=[_row_spec(dout), _vec_spec(), _vec_spec(), _vec_spec()],
        out_shape=[jax.ShapeDtypeStruct((NP, dout), jnp.float32),
                   jax.ShapeDtypeStruct((NP,), jnp.float32),
                   jax.ShapeDtypeStruct((NP,), jnp.float32),
                   jax.ShapeDtypeStruct((NP,), jnp.float32)],
    )(a0, a1, hprev, exs, den, b.reshape(1, -1), W,
      asrc.reshape(1, -1), adst.reshape(1, -1))


def _tc_pre4(a0, a1, hprev, exs, den, b, W, asrc, adst):
    return pl.pallas_call(
        _pre4_body,
        grid=(NP // RB,),
        in_specs=[_row_spec(128), _row_spec(128), _row_spec(128),
                  _vec_spec(), _vec_spec(), _full_spec((1, 128)),
                  _full_spec(W.shape),
                  _full_spec((1, 256)), _full_spec((1, 256))],
        out_specs=[_row_spec(128), _row_spec(128),
                   _vec_spec(), _vec_spec(), _vec_spec()],
        out_shape=[jax.ShapeDtypeStruct((NP, 128), jnp.float32),
                   jax.ShapeDtypeStruct((NP, 128), jnp.float32),
                   jax.ShapeDtypeStruct((NP,), jnp.float32),
                   jax.ShapeDtypeStruct((NP,), jnp.float32),
                   jax.ShapeDtypeStruct((NP,), jnp.float32)],
    )(a0, a1, hprev, exs, den, b.reshape(1, -1), W,
      asrc.reshape(1, -1), adst.reshape(1, -1))


def _tc_head(a0, a1, h0, h1, exs, den, b4, batchp, l1W, l1b, l2W, l2b):
    return pl.pallas_call(
        _head_body,
        grid=(NP // RB,),
        in_specs=[_row_spec(128), _row_spec(128), _row_spec(128),
                  _row_spec(128), _vec_spec(), _vec_spec(),
                  _full_spec((1, 256)), _vec_spec(),
                  _full_spec((256, 128)), _full_spec((1, 128)),
                  _full_spec((128, 10)), _full_spec((1, 10))],
        out_specs=pl.BlockSpec((BATCHES, 10), lambda i: (0, 0)),
        out_shape=jax.ShapeDtypeStruct((BATCHES, 10), jnp.float32),
        scratch_shapes=[pltpu.VMEM((BATCHES, 256), jnp.float32),
                        pltpu.VMEM((1, BATCHES), jnp.float32)],
    )(a0, a1, h0, h1, exs, den, b4.reshape(1, -1), batchp,
      l1W, l1b.reshape(1, -1), l2W, l2b.reshape(1, -1))


def _make_sc_layer(dout_b, edge_split):
    mesh = plsc.VectorSubcoreMesh(core_axis_name="c", subcore_axis_name="s",
                                  num_cores=2, num_subcores=16)
    n_h = 1 if edge_split else 2

    def body(srcp_h, dstp_h, ssrc_h, sdst_h, exself_h, zeros_h,
             *rest):
        h_tabs = rest[:n_h]
        out_h, den_out_h = rest[n_h], rest[n_h + 1]
        (src_sub, dst_sub, src2_sub, dst2_sub, a_sub, b_sub, den_sub,
         coef_sub, rows_v, sem,
         shared_ssrc, shared_sdst, shared_den, shared_out) = rest[n_h + 2:]

        c = lax.axis_index("c")
        s = lax.axis_index("s")
        soff = pl.multiple_of(s * STRIPE, STRIPE)

        @pl.when(s == 0)
        def _():
            pltpu.sync_copy(ssrc_h, shared_ssrc)

        @pl.when(s == 1)
        def _():
            pltpu.sync_copy(sdst_h, shared_sdst)

        @pl.when(s == 2)
        def _():
            pltpu.sync_copy(exself_h, shared_den)

        pltpu.sync_copy(zeros_h, shared_out.at[pl.ds(soff, STRIPE)])
        plsc.subcore_barrier()

        def load_chunk(off):
            pltpu.sync_copy(srcp_h.at[pl.ds(off, SUB)], src_sub)
            pltpu.sync_copy(dstp_h.at[pl.ds(off, SUB)], dst_sub)

            def mirror(k, _):
                i16 = src_sub[pl.ds(k * 16, 16)]
                d16 = dst_sub[pl.ds(k * 16, 16)]
                src2_sub[k // 8, pl.ds((k % 8) * 16, 16)] = i16
                dst2_sub[k // 8, pl.ds((k % 8) * 16, 16)] = d16
                return 0
            lax.fori_loop(0, SUB // 16, mirror, 0)
            for g in range(SUB // GRP):
                pltpu.sync_copy(shared_ssrc.at[src2_sub.at[g]],
                                a_sub.at[pl.ds(g * GRP, GRP)])
                pltpu.sync_copy(shared_sdst.at[dst2_sub.at[g]],
                                b_sub.at[pl.ds(g * GRP, GRP)])

        def compute_ex(off):
            def stepE(k, _):
                al = a_sub[pl.ds(k * 16, 16)] + b_sub[pl.ds(k * 16, 16)]
                al = jnp.where(al >= 0, al, 0.2 * al)
                ex = jnp.exp(al)
                eidx = off + k * 16 + lax.iota(jnp.int32, 16)
                coef_sub[pl.ds(k * 16, 16)] = jnp.where(eidx < E, ex, 0.0)
                return 0
            lax.fori_loop(0, SUB // 16, stepE, 0)

        ept_a = EP // 16

        def subA(sub, _):
            off = pl.multiple_of(s * ept_a + sub * SUB, SUB)
            load_chunk(off)
            compute_ex(off)
            for g in range(SUB // GRP):
                pltpu.sync_copy(coef_sub.at[pl.ds(g * GRP, GRP)],
                                shared_den.at[dst2_sub.at[g]], add=True)
            return 0
        lax.fori_loop(0, ept_a // SUB, subA, 0)
        plsc.subcore_barrier()

        @pl.when(c == 0)
        def _():
            pltpu.sync_copy(shared_den.at[pl.ds(soff, STRIPE)],
                            den_out_h.at[pl.ds(soff, STRIPE)])

        def phaseB(h_tab, base_b, ept_b):
            def subB(sub, _):
                off = pl.multiple_of(base_b + sub * SUB, SUB)
                load_chunk(off)
                compute_ex(off)
                for g in range(SUB // GRP):
                    pltpu.sync_copy(shared_den.at[dst2_sub.at[g]],
                                    den_sub.at[pl.ds(g * GRP, GRP)])

                def stepC(k, _):
                    ex16 = coef_sub[pl.ds(k * 16, 16)]
                    dn = den_sub[pl.ds(k * 16, 16)]
                    coef_sub[pl.ds(k * 16, 16)] = ex16 / (dn + 1e-16)
                    return 0
                lax.fori_loop(0, SUB // 16, stepC, 0)

                for g in range(SUB // GRP):
                    pltpu.async_copy(h_tab.at[src2_sub.at[g]],
                                     rows_v, sem).wait()

                    def stepS(e, _):
                        cfb = plsc.load_gather(
                            coef_sub,
                            [jnp.full((16,), g * GRP + e, jnp.int32)])
                        for cb in range(dout_b // 16):
                            v = rows_v[e, pl.ds(cb * 16, 16)]
                            rows_v[e, pl.ds(cb * 16, 16)] = v * cfb
                        return 0
                    lax.fori_loop(0, GRP, stepS, 0)

                    pltpu.sync_copy(rows_v, shared_out.at[dst2_sub.at[g]],
                                    add=True)
                return 0
            lax.fori_loop(0, ept_b // SUB, subB, 0)

        if edge_split:
            ept_b = EP // 32
            base_b = c * (EP // 2) + s * ept_b
            phaseB(h_tabs[0], base_b, ept_b)
        else:
            ept_b = EP // 16
            base_b = s * ept_b

            @pl.when(c == 0)
            def _():
                phaseB(h_tabs[0], base_b, ept_b)

            @pl.when(c == 1)
            def _():
                phaseB(h_tabs[1], base_b, ept_b)

        plsc.subcore_barrier()
        for cc in range(2):
            @pl.when(c == cc)
            def _(cc=cc):
                pltpu.sync_copy(
                    shared_out.at[pl.ds(soff, STRIPE)],
                    out_h.at[cc, pl.ds(soff, STRIPE)])

    scratch = [
        pltpu.VMEM((SUB,), jnp.int32),
        pltpu.VMEM((SUB,), jnp.int32),
        pltpu.VMEM((SUB // GRP, GRP), jnp.int32),
        pltpu.VMEM((SUB // GRP, GRP), jnp.int32),
        pltpu.VMEM((SUB,), jnp.float32),
        pltpu.VMEM((SUB,), jnp.float32),
        pltpu.VMEM((SUB,), jnp.float32),
        pltpu.VMEM((SUB,), jnp.float32),
        pltpu.VMEM((GRP, dout_b), jnp.float32),
        pltpu.SemaphoreType.DMA,
        pltpu.VMEM_SHARED((NP,), jnp.float32),
        pltpu.VMEM_SHARED((NP,), jnp.float32),
        pltpu.VMEM_SHARED((NP,), jnp.float32),
        pltpu.VMEM_SHARED((NP, dout_b), jnp.float32),
    ]

    out_type = (jax.ShapeDtypeStruct((2, NP, dout_b), jnp.float32),
                jax.ShapeDtypeStruct((NP,), jnp.float32))

    return pl.kernel(body, out_type=out_type, mesh=mesh,
                     scratch_types=scratch,
                     compiler_params=pltpu.CompilerParams(
                         needs_layout_passes=False,
                         use_tc_tiling_on_sc=False))


def _sc_run(dout, edge_split, srcp, dstp, ssrc, sdst, exself, htabs):
    dout_b = dout if edge_split else dout // 2
    zeros = jnp.zeros((STRIPE, dout_b), jnp.float32)
    k = _make_sc_layer(dout_b, edge_split)
    return k(srcp, dstp, ssrc, sdst, exself, zeros, *htabs)


def kernel(x, edge_index, batch,
           W1, att_src1, att_dst1, b1,
           W2, att_src2, att_dst2, b2,
           W3, att_src3, att_dst3, b3,
           W4, att_src4, att_dst4, b4,
           lin1_W, lin1_b, lin2_W, lin2_b):
    srcp = jnp.concatenate(
        [edge_index[0], jnp.zeros((EP - E,), jnp.int32)])
    dstp = jnp.concatenate(
        [edge_index[1], jnp.zeros((EP - E,), jnp.int32)])
    xp = jnp.pad(x, ((0, NP - N), (0, 0)))
    batchp = jnp.pad(batch, (0, NP - N), constant_values=BATCHES)

    h1, ss1, sd1, exs1 = _tc_pre1(xp, W1, att_src1, att_dst1, 32)
    agg1, den1 = _sc_run(32, True, srcp, dstp, ss1, sd1, exs1, (h1,))

    h2, ss2, sd2, exs2 = _tc_pre_sum(
        agg1[0], agg1[1], h1, exs1, den1, b1, W2, att_src2, att_dst2, 32, 64)
    agg2, den2 = _sc_run(64, True, srcp, dstp, ss2, sd2, exs2, (h2,))

    h3, ss3, sd3, exs3 = _tc_pre_sum(
        agg2[0], agg2[1], h2, exs2, den2, b2, W3, att_src3, att_dst3, 64, 128)
    agg3, den3 = _sc_run(128, True, srcp, dstp, ss3, sd3, exs3, (h3,))

    h4a, h4b, ss4, sd4, exs4 = _tc_pre4(
        agg3[0], agg3[1], h3, exs3, den3, b3, W4, att_src4, att_dst4)
    agg4, den4 = _sc_run(256, False, srcp, dstp, ss4, sd4, exs4,
                         (h4a, h4b))

    return _tc_head(agg4[0], agg4[1], h4a, h4b, exs4, den4, b4, batchp,
                    lin1_W, lin1_b, lin2_W, lin2_b)


# speedup vs baseline: 22.5075x; 1.4393x over previous
import functools

import jax
import jax.numpy as jnp
from jax import lax
from jax.experimental import pallas as pl
from jax.experimental.pallas import tpu as pltpu
from jax.experimental.pallas import tpu_sc as plsc

N = 10000
NP = 10240
E = 320000
EP = 327680
BATCHES = 128
SUB = 2048
GRP = 128
RB = 1024
STRIPE = NP // 16


def _elu(v):
    return jnp.where(v > 0, v, jnp.exp(jnp.minimum(v, 0.0)) - 1.0)


def _scores(h, asrc_ref, adst_ref):
    ss = jnp.sum(h * asrc_ref[...], axis=1)
    sd = jnp.sum(h * adst_ref[...], axis=1)
    al = ss + sd
    exself = jnp.exp(jnp.where(al >= 0, al, 0.2 * al))
    return ss, sd, exself


def _pre1_body(x_ref, W_ref, asrc_ref, adst_ref,
               h_ref, ssrc_ref, sdst_ref, exself_ref):
    h = jnp.dot(x_ref[...], W_ref[...], preferred_element_type=jnp.float32)
    h_ref[...] = h
    ss, sd, exself = _scores(h, asrc_ref, adst_ref)
    ssrc_ref[...] = ss
    sdst_ref[...] = sd
    exself_ref[...] = exself


def _pre_sum_body(a0_ref, a1_ref, hprev_ref, exs_ref, den_ref, b_ref,
                  W_ref, asrc_ref, adst_ref,
                  h_ref, ssrc_ref, sdst_ref, exself_ref):
    cs = (exs_ref[...] / (den_ref[...] + 1e-16)).reshape(-1, 1)
    x = _elu(a0_ref[...] + a1_ref[...] + hprev_ref[...] * cs + b_ref[...])
    h = jnp.dot(x, W_ref[...], preferred_element_type=jnp.float32)
    h_ref[...] = h
    ss, sd, exself = _scores(h, asrc_ref, adst_ref)
    ssrc_ref[...] = ss
    sdst_ref[...] = sd
    exself_ref[...] = exself


def _pre4_body(a0_ref, a1_ref, hprev_ref, exs_ref, den_ref, b_ref,
               W_ref, asrc_ref, adst_ref,
               h0_ref, h1_ref, ssrc_ref, sdst_ref, exself_ref):
    cs = (exs_ref[...] / (den_ref[...] + 1e-16)).reshape(-1, 1)
    x = _elu(a0_ref[...] + a1_ref[...] + hprev_ref[...] * cs + b_ref[...])
    h = jnp.dot(x, W_ref[...], preferred_element_type=jnp.float32)
    h0_ref[...] = h[:, :128]
    h1_ref[...] = h[:, 128:]
    ss, sd, exself = _scores(h, asrc_ref, adst_ref)
    ssrc_ref[...] = ss
    sdst_ref[...] = sd
    exself_ref[...] = exself


def _head_body(a0_ref, a1_ref, h0_ref, h1_ref, exs_ref, den_ref, b_ref,
               batch_ref, l1W_ref, l1b_ref, l2W_ref, l2b_ref,
               out_ref, sums_ref, counts_ref):
    i = pl.program_id(0)

    @pl.when(i == 0)
    def _():
        sums_ref[...] = jnp.zeros_like(sums_ref)
        counts_ref[...] = jnp.zeros_like(counts_ref)

    cs = (exs_ref[...] / (den_ref[...] + 1e-16)).reshape(-1, 1)
    agg = jnp.concatenate([a0_ref[...], a1_ref[...]], axis=1)
    hprev = jnp.concatenate([h0_ref[...], h1_ref[...]], axis=1)
    x = _elu(agg + hprev * cs + b_ref[...])
    bb = batch_ref[...].reshape(RB, 1)
    P = (bb == lax.broadcasted_iota(jnp.int32, (RB, BATCHES), 1)
         ).astype(jnp.float32)
    sums_ref[...] += lax.dot_general(
        P, x, (((0,), (0,)), ((), ())), preferred_element_type=jnp.float32)
    counts_ref[...] += jnp.sum(P, axis=0).reshape(1, BATCHES)

    @pl.when(i == pl.num_programs(0) - 1)
    def _():
        counts = jnp.maximum(counts_ref[...].reshape(BATCHES, 1), 1.0)
        pooled = sums_ref[...] / counts
        hh = _elu(jnp.dot(pooled, l1W_ref[...],
                          preferred_element_type=jnp.float32) + l1b_ref[...])
        out_ref[...] = jnp.dot(hh, l2W_ref[...],
                               preferred_element_type=jnp.float32) + l2b_ref[...]


def _row_spec(width):
    return pl.BlockSpec((RB, width), lambda i: (i, 0))


def _vec_spec():
    return pl.BlockSpec((RB,), lambda i: (i,))


def _full_spec(shape):
    return pl.BlockSpec(shape, lambda i: tuple(0 for _ in shape))


def _tc_pre1(xp, W, asrc, adst, dout):
    return pl.pallas_call(
        _pre1_body,
        grid=(NP // RB,),
        in_specs=[_row_spec(128), _full_spec(W.shape),
                  _full_spec((1, dout)), _full_spec((1, dout))],
        out_specs=[_row_spec(dout), _vec_spec(), _vec_spec(), _vec_spec()],
        out_shape=[jax.ShapeDtypeStruct((NP, dout), jnp.float32),
                   jax.ShapeDtypeStruct((NP,), jnp.float32),
                   jax.ShapeDtypeStruct((NP,), jnp.float32),
                   jax.ShapeDtypeStruct((NP,), jnp.float32)],
    )(xp, W, asrc.reshape(1, -1), adst.reshape(1, -1))


def _tc_pre_sum(a0, a1, hprev, exs, den, b, W, asrc, adst, din, dout):
    return pl.pallas_call(
        _pre_sum_body,
        grid=(NP // RB,),
        in_specs=[_row_spec(din), _row_spec(din), _row_spec(din),
                  _vec_spec(), _vec_spec(), _full_spec((1, din)),
                  _full_spec(W.shape),
                  _full_spec((1, dout)), _full_spec((1, dout))],
        out_specs=[_row_spec(dout), _vec_spec(), _vec_spec(), _vec_spec()],
        out_shape=[jax.ShapeDtypeStruct((NP, dout), jnp.float32),
                   jax.ShapeDtypeStruct((NP,), jnp.float32),
                   jax.ShapeDtypeStruct((NP,), jnp.float32),
                   jax.ShapeDtypeStruct((NP,), jnp.float32)],
    )(a0, a1, hprev, exs, den, b.reshape(1, -1), W,
      asrc.reshape(1, -1), adst.reshape(1, -1))


def _tc_pre4(a0, a1, hprev, exs, den, b, W, asrc, adst):
    return pl.pallas_call(
        _pre4_body,
        grid=(NP // RB,),
        in_specs=[_row_spec(128), _row_spec(128), _row_spec(128),
                  _vec_spec(), _vec_spec(), _full_spec((1, 128)),
                  _full_spec(W.shape),
                  _full_spec((1, 256)), _full_spec((1, 256))],
        out_specs=[_row_spec(128), _row_spec(128),
                   _vec_spec(), _vec_spec(), _vec_spec()],
        out_shape=[jax.ShapeDtypeStruct((NP, 128), jnp.float32),
                   jax.ShapeDtypeStruct((NP, 128), jnp.float32),
                   jax.ShapeDtypeStruct((NP,), jnp.float32),
                   jax.ShapeDtypeStruct((NP,), jnp.float32),
                   jax.ShapeDtypeStruct((NP,), jnp.float32)],
    )(a0, a1, hprev, exs, den, b.reshape(1, -1), W,
      asrc.reshape(1, -1), adst.reshape(1, -1))


def _tc_head(a0, a1, h0, h1, exs, den, b4, batchp, l1W, l1b, l2W, l2b):
    return pl.pallas_call(
        _head_body,
        grid=(NP // RB,),
        in_specs=[_row_spec(128), _row_spec(128), _row_spec(128),
                  _row_spec(128), _vec_spec(), _vec_spec(),
                  _full_spec((1, 256)), _vec_spec(),
                  _full_spec((256, 128)), _full_spec((1, 128)),
                  _full_spec((128, 10)), _full_spec((1, 10))],
        out_specs=pl.BlockSpec((BATCHES, 10), lambda i: (0, 0)),
        out_shape=jax.ShapeDtypeStruct((BATCHES, 10), jnp.float32),
        scratch_shapes=[pltpu.VMEM((BATCHES, 256), jnp.float32),
                        pltpu.VMEM((1, BATCHES), jnp.float32)],
    )(a0, a1, h0, h1, exs, den, b4.reshape(1, -1), batchp,
      l1W, l1b.reshape(1, -1), l2W, l2b.reshape(1, -1))


def _make_sc_layer(dout_b, edge_split, G):
    NG = SUB // G
    mesh = plsc.VectorSubcoreMesh(core_axis_name="c", subcore_axis_name="s",
                                  num_cores=2, num_subcores=16)
    n_h = 1 if edge_split else 2

    def body(srcp2_h, dstp2_h, ssrc_h, sdst_h, exself_h, zeros_h,
             *rest):
        h_tabs = rest[:n_h]
        out_h, den_out_h = rest[n_h], rest[n_h + 1]
        (src2_sub, dst2_sub, a_sub, b_sub, den_sub, coef_sub,
         rows0, rows1, sem_l, sem_a, sem_b, gsem0, gsem1, ssem0, ssem1,
         shared_ssrc, shared_sdst, shared_den, shared_out) = rest[n_h + 2:]
        rows = (rows0, rows1)
        gsem = (gsem0, gsem1)
        ssem = (ssem0, ssem1)

        c = lax.axis_index("c")
        s = lax.axis_index("s")
        soff = pl.multiple_of(s * STRIPE, STRIPE)

        @pl.when(s == 0)
        def _():
            pltpu.sync_copy(ssrc_h, shared_ssrc)

        @pl.when(s == 1)
        def _():
            pltpu.sync_copy(sdst_h, shared_sdst)

        @pl.when(s == 2)
        def _():
            pltpu.sync_copy(exself_h, shared_den)

        pltpu.sync_copy(zeros_h, shared_out.at[pl.ds(soff, STRIPE)])
        plsc.subcore_barrier()

        def load_idx(off):
            offr = pl.multiple_of(off // G, NG)
            d1 = pltpu.async_copy(srcp2_h.at[pl.ds(offr, NG)], src2_sub,
                                  sem_l)
            d2 = pltpu.async_copy(dstp2_h.at[pl.ds(offr, NG)], dst2_sub,
                                  sem_l)
            d1.wait()
            d2.wait()
            descs = []
            for g in range(NG):
                descs.append(pltpu.async_copy(
                    shared_ssrc.at[src2_sub.at[g]],
                    a_sub.at[pl.ds(g * G, G)], sem_a))
                descs.append(pltpu.async_copy(
                    shared_sdst.at[dst2_sub.at[g]],
                    b_sub.at[pl.ds(g * G, G)], sem_b))
            for d in descs:
                d.wait()

        def compute_ex(off):
            def stepE(k, _):
                al = a_sub[pl.ds(k * 16, 16)] + b_sub[pl.ds(k * 16, 16)]
                al = jnp.where(al >= 0, al, 0.2 * al)
                ex = jnp.exp(al)
                eidx = off + k * 16 + lax.iota(jnp.int32, 16)
                coef_sub[pl.ds(k * 16, 16)] = jnp.where(eidx < E, ex, 0.0)
                return 0
            lax.fori_loop(0, SUB // 16, stepE, 0)

        ept_a = EP // 16

        def subA(sub, _):
            off = pl.multiple_of(s * ept_a + sub * SUB, SUB)
            load_idx(off)
            compute_ex(off)
            descs = [pltpu.async_copy(coef_sub.at[pl.ds(g * G, G)],
                                      shared_den.at[dst2_sub.at[g]],
                                      sem_b, add=True)
                     for g in range(NG)]
            for d in descs:
                d.wait()
            return 0
        lax.fori_loop(0, ept_a // SUB, subA, 0)
        plsc.subcore_barrier()

        @pl.when(c == 0)
        def _():
            pltpu.sync_copy(shared_den.at[pl.ds(soff, STRIPE)],
                            den_out_h.at[pl.ds(soff, STRIPE)])

        def phaseB(h_tab, base_b, ept_b):
            def fire_g(g):
                return pltpu.async_copy(h_tab.at[src2_sub.at[g]],
                                        rows[g % 2], gsem[g % 2])

            def fire_s(g):
                return pltpu.async_copy(rows[g % 2],
                                        shared_out.at[dst2_sub.at[g]],
                                        ssem[g % 2], add=True)

            def scale(g):
                def stepS(e, _):
                    cfb = plsc.load_gather(
                        coef_sub, [jnp.full((16,), g * G + e, jnp.int32)])
                    r = rows[g % 2]
                    for cb in range(dout_b // 16):
                        v = r[e, pl.ds(cb * 16, 16)]
                        r[e, pl.ds(cb * 16, 16)] = v * cfb
                    return 0
                lax.fori_loop(0, G, stepS, 0)

            def subB(sub, _):
                off = pl.multiple_of(base_b + sub * SUB, SUB)
                load_idx(off)
                compute_ex(off)
                descs = [pltpu.async_copy(shared_den.at[dst2_sub.at[g]],
                                          den_sub.at[pl.ds(g * G, G)],
                                          sem_a)
                         for g in range(NG)]
                for d in descs:
                    d.wait()

                def stepC(k, _):
                    ex16 = coef_sub[pl.ds(k * 16, 16)]
                    dn = den_sub[pl.ds(k * 16, 16)]
                    coef_sub[pl.ds(k * 16, 16)] = ex16 / (dn + 1e-16)
                    return 0
                lax.fori_loop(0, SUB // 16, stepC, 0)

                gd = [None] * NG
                sd = [None] * NG
                gd[0] = fire_g(0)
                for g in range(NG):
                    if g + 1 < NG:
                        if g >= 1:
                            sd[g - 1].wait()
                        gd[g + 1] = fire_g(g + 1)
                    gd[g].wait()
                    scale(g)
                    sd[g] = fire_s(g)
                if NG >= 2:
                    sd[NG - 2].wait()
                sd[NG - 1].wait()
                return 0
            lax.fori_loop(0, ept_b // SUB, subB, 0)

        if edge_split:
            ept_b = EP // 32
            base_b = c * (EP // 2) + s * ept_b
            phaseB(h_tabs[0], base_b, ept_b)
        else:
            ept_b = EP // 16
            base_b = s * ept_b

            @pl.when(c == 0)
            def _():
                phaseB(h_tabs[0], base_b, ept_b)

            @pl.when(c == 1)
            def _():
                phaseB(h_tabs[1], base_b, ept_b)

        plsc.subcore_barrier()
        for cc in range(2):
            @pl.when(c == cc)
            def _(cc=cc):
                pltpu.sync_copy(
                    shared_out.at[pl.ds(soff, STRIPE)],
                    out_h.at[cc, pl.ds(soff, STRIPE)])

    scratch = [
        pltpu.VMEM((NG, G), jnp.int32),
        pltpu.VMEM((NG, G), jnp.int32),
        pltpu.VMEM((SUB,), jnp.float32),
        pltpu.VMEM((SUB,), jnp.float32),
        pltpu.VMEM((SUB,), jnp.float32),
        pltpu.VMEM((SUB,), jnp.float32),
        pltpu.VMEM((G, dout_b), jnp.float32),
        pltpu.VMEM((G, dout_b), jnp.float32),
        pltpu.SemaphoreType.DMA,
        pltpu.SemaphoreType.DMA,
        pltpu.SemaphoreType.DMA,
        pltpu.SemaphoreType.DMA,
        pltpu.SemaphoreType.DMA,
        pltpu.SemaphoreType.DMA,
        pltpu.SemaphoreType.DMA,
        pltpu.VMEM_SHARED((NP,), jnp.float32),
        pltpu.VMEM_SHARED((NP,), jnp.float32),
        pltpu.VMEM_SHARED((NP,), jnp.float32),
        pltpu.VMEM_SHARED((NP, dout_b), jnp.float32),
    ]

    out_type = (jax.ShapeDtypeStruct((2, NP, dout_b), jnp.float32),
                jax.ShapeDtypeStruct((NP,), jnp.float32))

    return pl.kernel(body, out_type=out_type, mesh=mesh,
                     scratch_types=scratch,
                     compiler_params=pltpu.CompilerParams(
                         needs_layout_passes=False,
                         use_tc_tiling_on_sc=False))


def _sc_run(dout, edge_split, srcp, dstp, ssrc, sdst, exself, htabs):
    dout_b = dout if edge_split else dout // 2
    G = 128 if dout_b <= 64 else 64
    zeros = jnp.zeros((STRIPE, dout_b), jnp.float32)
    k = _make_sc_layer(dout_b, edge_split, G)
    return k(srcp.reshape(EP // G, G), dstp.reshape(EP // G, G),
             ssrc, sdst, exself, zeros, *htabs)


def kernel(x, edge_index, batch,
           W1, att_src1, att_dst1, b1,
           W2, att_src2, att_dst2, b2,
           W3, att_src3, att_dst3, b3,
           W4, att_src4, att_dst4, b4,
           lin1_W, lin1_b, lin2_W, lin2_b):
    srcp = jnp.concatenate(
        [edge_index[0], jnp.zeros((EP - E,), jnp.int32)])
    dstp = jnp.concatenate(
        [edge_index[1], jnp.zeros((EP - E,), jnp.int32)])
    xp = jnp.pad(x, ((0, NP - N), (0, 0)))
    batchp = jnp.pad(batch, (0, NP - N), constant_values=BATCHES)

    h1, ss1, sd1, exs1 = _tc_pre1(xp, W1, att_src1, att_dst1, 32)
    agg1, den1 = _sc_run(32, True, srcp, dstp, ss1, sd1, exs1, (h1,))

    h2, ss2, sd2, exs2 = _tc_pre_sum(
        agg1[0], agg1[1], h1, exs1, den1, b1, W2, att_src2, att_dst2, 32, 64)
    agg2, den2 = _sc_run(64, True, srcp, dstp, ss2, sd2, exs2, (h2,))

    h3, ss3, sd3, exs3 = _tc_pre_sum(
        agg2[0], agg2[1], h2, exs2, den2, b2, W3, att_src3, att_dst3, 64, 128)
    agg3, den3 = _sc_run(128, True, srcp, dstp, ss3, sd3, exs3, (h3,))

    h4a, h4b, ss4, sd4, exs4 = _tc_pre4(
        agg3[0], agg3[1], h3, exs3, den3, b3, W4, att_src4, att_dst4)
    agg4, den4 = _sc_run(256, False, srcp, dstp, ss4, sd4, exs4,
                         (h4a, h4b))

    return _tc_head(agg4[0], agg4[1], h4a, h4b, exs4, den4, b4, batchp,
                    lin1_W, lin1_b, lin2_W, lin2_b)


# speedup vs baseline: 23.2525x; 1.0331x over previous
import functools

import jax
import jax.numpy as jnp
from jax import lax
from jax.experimental import pallas as pl
from jax.experimental.pallas import tpu as pltpu
from jax.experimental.pallas import tpu_sc as plsc

N = 10000
NP = 10240
E = 320000
EP = 327680
BATCHES = 128
SUB = 2048
GRP = 128
RB = 1024
STRIPE = NP // 16


def _elu(v):
    return jnp.where(v > 0, v, jnp.exp(jnp.minimum(v, 0.0)) - 1.0)


def _scores(h, asrc_ref, adst_ref):
    ss = jnp.sum(h * asrc_ref[...], axis=1)
    sd = jnp.sum(h * adst_ref[...], axis=1)
    al = ss + sd
    exself = jnp.exp(jnp.where(al >= 0, al, 0.2 * al))
    return ss, sd, exself


def _pre1_body(x_ref, W_ref, asrc_ref, adst_ref,
               h_ref, ssrc_ref, sdst_ref, exself_ref):
    h = jnp.dot(x_ref[...], W_ref[...], preferred_element_type=jnp.float32)
    h_ref[...] = h
    ss, sd, exself = _scores(h, asrc_ref, adst_ref)
    ssrc_ref[...] = ss
    sdst_ref[...] = sd
    exself_ref[...] = exself


def _pre_sum_body(a0_ref, a1_ref, hprev_ref, exs_ref, den_ref, b_ref,
                  W_ref, asrc_ref, adst_ref,
                  h_ref, ssrc_ref, sdst_ref, exself_ref):
    cs = (exs_ref[...] / (den_ref[...] + 1e-16)).reshape(-1, 1)
    x = _elu(a0_ref[...] + a1_ref[...] + hprev_ref[...] * cs + b_ref[...])
    h = jnp.dot(x, W_ref[...], preferred_element_type=jnp.float32)
    h_ref[...] = h
    ss, sd, exself = _scores(h, asrc_ref, adst_ref)
    ssrc_ref[...] = ss
    sdst_ref[...] = sd
    exself_ref[...] = exself


def _pre4_body(a0_ref, a1_ref, hprev_ref, exs_ref, den_ref, b_ref,
               W_ref, asrc_ref, adst_ref,
               h0_ref, h1_ref, ssrc_ref, sdst_ref, exself_ref):
    cs = (exs_ref[...] / (den_ref[...] + 1e-16)).reshape(-1, 1)
    x = _elu(a0_ref[...] + a1_ref[...] + hprev_ref[...] * cs + b_ref[...])
    h = jnp.dot(x, W_ref[...], preferred_element_type=jnp.float32)
    h0_ref[...] = h[:, :128]
    h1_ref[...] = h[:, 128:]
    ss, sd, exself = _scores(h, asrc_ref, adst_ref)
    ssrc_ref[...] = ss
    sdst_ref[...] = sd
    exself_ref[...] = exself


def _head_body(a0_ref, a1_ref, h0_ref, h1_ref, exs_ref, den_ref, b_ref,
               batch_ref, l1W_ref, l1b_ref, l2W_ref, l2b_ref,
               out_ref, sums_ref, counts_ref):
    i = pl.program_id(0)

    @pl.when(i == 0)
    def _():
        sums_ref[...] = jnp.zeros_like(sums_ref)
        counts_ref[...] = jnp.zeros_like(counts_ref)

    cs = (exs_ref[...] / (den_ref[...] + 1e-16)).reshape(-1, 1)
    agg = jnp.concatenate([a0_ref[...], a1_ref[...]], axis=1)
    hprev = jnp.concatenate([h0_ref[...], h1_ref[...]], axis=1)
    x = _elu(agg + hprev * cs + b_ref[...])
    bb = batch_ref[...].reshape(RB, 1)
    P = (bb == lax.broadcasted_iota(jnp.int32, (RB, BATCHES), 1)
         ).astype(jnp.float32)
    sums_ref[...] += lax.dot_general(
        P, x, (((0,), (0,)), ((), ())), preferred_element_type=jnp.float32)
    counts_ref[...] += jnp.sum(P, axis=0).reshape(1, BATCHES)

    @pl.when(i == pl.num_programs(0) - 1)
    def _():
        counts = jnp.maximum(counts_ref[...].reshape(BATCHES, 1), 1.0)
        pooled = sums_ref[...] / counts
        hh = _elu(jnp.dot(pooled, l1W_ref[...],
                          preferred_element_type=jnp.float32) + l1b_ref[...])
        out_ref[...] = jnp.dot(hh, l2W_ref[...],
                               preferred_element_type=jnp.float32) + l2b_ref[...]


def _row_spec(width):
    return pl.BlockSpec((RB, width), lambda i: (i, 0))


def _vec_spec():
    return pl.BlockSpec((RB,), lambda i: (i,))


def _full_spec(shape):
    return pl.BlockSpec(shape, lambda i: tuple(0 for _ in shape))


def _tc_pre1(xp, W, asrc, adst, dout):
    return pl.pallas_call(
        _pre1_body,
        grid=(NP // RB,),
        in_specs=[_row_spec(128), _full_spec(W.shape),
                  _full_spec((1, dout)), _full_spec((1, dout))],
        out_specs=[_row_spec(dout), _vec_spec(), _vec_spec(), _vec_spec()],
        out_shape=[jax.ShapeDtypeStruct((NP, dout), jnp.float32),
                   jax.ShapeDtypeStruct((NP,), jnp.float32),
                   jax.ShapeDtypeStruct((NP,), jnp.float32),
                   jax.ShapeDtypeStruct((NP,), jnp.float32)],
    )(xp, W, asrc.reshape(1, -1), adst.reshape(1, -1))


def _tc_pre_sum(a0, a1, hprev, exs, den, b, W, asrc, adst, din, dout):
    return pl.pallas_call(
        _pre_sum_body,
        grid=(NP // RB,),
        in_specs=[_row_spec(din), _row_spec(din), _row_spec(din),
                  _vec_spec(), _vec_spec(), _full_spec((1, din)),
                  _full_spec(W.shape),
                  _full_spec((1, dout)), _full_spec((1, dout))],
        out_specs=[_row_spec(dout), _vec_spec(), _vec_spec(), _vec_spec()],
        out_shape=[jax.ShapeDtypeStruct((NP, dout), jnp.float32),
                   jax.ShapeDtypeStruct((NP,), jnp.float32),
                   jax.ShapeDtypeStruct((NP,), jnp.float32),
                   jax.ShapeDtypeStruct((NP,), jnp.float32)],
    )(a0, a1, hprev, exs, den, b.reshape(1, -1), W,
      asrc.reshape(1, -1), adst.reshape(1, -1))


def _tc_pre4(a0, a1, hprev, exs, den, b, W, asrc, adst):
    return pl.pallas_call(
        _pre4_body,
        grid=(NP // RB,),
        in_specs=[_row_spec(128), _row_spec(128), _row_spec(128),
                  _vec_spec(), _vec_spec(), _full_spec((1, 128)),
                  _full_spec(W.shape),
                  _full_spec((1, 256)), _full_spec((1, 256))],
        out_specs=[_row_spec(128), _row_spec(128),
                   _vec_spec(), _vec_spec(), _vec_spec()],
        out_shape=[jax.ShapeDtypeStruct((NP, 128), jnp.float32),
                   jax.ShapeDtypeStruct((NP, 128), jnp.float32),
                   jax.ShapeDtypeStruct((NP,), jnp.float32),
                   jax.ShapeDtypeStruct((NP,), jnp.float32),
                   jax.ShapeDtypeStruct((NP,), jnp.float32)],
    )(a0, a1, hprev, exs, den, b.reshape(1, -1), W,
      asrc.reshape(1, -1), adst.reshape(1, -1))


def _tc_head(a0, a1, h0, h1, exs, den, b4, batchp, l1W, l1b, l2W, l2b):
    return pl.pallas_call(
        _head_body,
        grid=(NP // RB,),
        in_specs=[_row_spec(128), _row_spec(128), _row_spec(128),
                  _row_spec(128), _vec_spec(), _vec_spec(),
                  _full_spec((1, 256)), _vec_spec(),
                  _full_spec((256, 128)), _full_spec((1, 128)),
                  _full_spec((128, 10)), _full_spec((1, 10))],
        out_specs=pl.BlockSpec((BATCHES, 10), lambda i: (0, 0)),
        out_shape=jax.ShapeDtypeStruct((BATCHES, 10), jnp.float32),
        scratch_shapes=[pltpu.VMEM((BATCHES, 256), jnp.float32),
                        pltpu.VMEM((1, BATCHES), jnp.float32)],
    )(a0, a1, h0, h1, exs, den, b4.reshape(1, -1), batchp,
      l1W, l1b.reshape(1, -1), l2W, l2b.reshape(1, -1))


def _make_sc_layer(dout_b, edge_split, G):
    NG = SUB // G
    mesh = plsc.VectorSubcoreMesh(core_axis_name="c", subcore_axis_name="s",
                                  num_cores=2, num_subcores=16)
    n_h = 1 if edge_split else 2

    def body(srcp2_h, dstp2_h, ssrc_h, sdst_h, exself_h, zeros_h,
             *rest):
        h_tabs = rest[:n_h]
        out_h, den_out_h, ex_out_h = (rest[n_h], rest[n_h + 1],
                                      rest[n_h + 2])
        (src2_sub, dst2_sub, a_sub, b_sub, den_sub, coef_sub,
         rows0, rows1, sem_l, sem_a, sem_b, gsem0, gsem1, ssem0, ssem1,
         shared_ssrc, shared_sdst, shared_den, shared_out) = rest[n_h + 3:]
        rows = (rows0, rows1)
        gsem = (gsem0, gsem1)
        ssem = (ssem0, ssem1)

        c = lax.axis_index("c")
        s = lax.axis_index("s")
        soff = pl.multiple_of(s * STRIPE, STRIPE)

        @pl.when(s == 0)
        def _():
            pltpu.sync_copy(ssrc_h, shared_ssrc)

        @pl.when(s == 1)
        def _():
            pltpu.sync_copy(sdst_h, shared_sdst)

        @pl.when(s == 2)
        def _():
            pltpu.sync_copy(exself_h, shared_den)

        pltpu.sync_copy(zeros_h, shared_out.at[pl.ds(soff, STRIPE)])
        plsc.subcore_barrier()

        def load_blocks(off):
            offr = pl.multiple_of(off // G, NG)
            d1 = pltpu.async_copy(srcp2_h.at[pl.ds(offr, NG)], src2_sub,
                                  sem_l)
            d2 = pltpu.async_copy(dstp2_h.at[pl.ds(offr, NG)], dst2_sub,
                                  sem_l)
            return d1, d2

        def load_idx(off):
            d1, d2 = load_blocks(off)
            d1.wait()
            d2.wait()
            descs = []
            for g in range(NG):
                descs.append(pltpu.async_copy(
                    shared_ssrc.at[src2_sub.at[g]],
                    a_sub.at[pl.ds(g * G, G)], sem_a))
                descs.append(pltpu.async_copy(
                    shared_sdst.at[dst2_sub.at[g]],
                    b_sub.at[pl.ds(g * G, G)], sem_b))
            for d in descs:
                d.wait()

        def compute_ex(off):
            def stepE(k, _):
                al = a_sub[pl.ds(k * 16, 16)] + b_sub[pl.ds(k * 16, 16)]
                al = jnp.where(al >= 0, al, 0.2 * al)
                ex = jnp.exp(al)
                eidx = off + k * 16 + lax.iota(jnp.int32, 16)
                coef_sub[pl.ds(k * 16, 16)] = jnp.where(eidx < E, ex, 0.0)
                return 0
            lax.fori_loop(0, SUB // 16, stepE, 0)

        ept_a = EP // 16

        def subA(sub, _):
            off = pl.multiple_of(s * ept_a + sub * SUB, SUB)
            load_idx(off)
            compute_ex(off)
            descs = [pltpu.async_copy(coef_sub.at[pl.ds(g * G, G)],
                                      shared_den.at[dst2_sub.at[g]],
                                      sem_b, add=True)
                     for g in range(NG)]
            descs.append(pltpu.async_copy(
                coef_sub, ex_out_h.at[c, pl.ds(off, SUB)], sem_a))
            for d in descs:
                d.wait()
            return 0
        lax.fori_loop(0, ept_a // SUB, subA, 0)
        plsc.subcore_barrier()

        @pl.when(c == 0)
        def _():
            pltpu.sync_copy(shared_den.at[pl.ds(soff, STRIPE)],
                            den_out_h.at[pl.ds(soff, STRIPE)])

        def phaseB(h_tab, base_b, ept_b):
            def fire_g(g):
                return pltpu.async_copy(h_tab.at[src2_sub.at[g]],
                                        rows[g % 2], gsem[g % 2])

            def fire_s(g):
                return pltpu.async_copy(rows[g % 2],
                                        shared_out.at[dst2_sub.at[g]],
                                        ssem[g % 2], add=True)

            def scale(g):
                def stepS(e, _):
                    cfb = plsc.load_gather(
                        coef_sub, [jnp.full((16,), g * G + e, jnp.int32)])
                    r = rows[g % 2]
                    for cb in range(dout_b // 16):
                        v = r[e, pl.ds(cb * 16, 16)]
                        r[e, pl.ds(cb * 16, 16)] = v * cfb
                    return 0
                lax.fori_loop(0, G, stepS, 0)

            def subB(sub, _):
                off = pl.multiple_of(base_b + sub * SUB, SUB)
                d1, d2 = load_blocks(off)
                dex = pltpu.async_copy(ex_out_h.at[c, pl.ds(off, SUB)],
                                       coef_sub, sem_b)
                d1.wait()
                d2.wait()
                descs = [pltpu.async_copy(shared_den.at[dst2_sub.at[g]],
                                          den_sub.at[pl.ds(g * G, G)],
                                          sem_a)
                         for g in range(NG)]
                dex.wait()
                for d in descs:
                    d.wait()

                def stepC(k, _):
                    ex16 = coef_sub[pl.ds(k * 16, 16)]
                    dn = den_sub[pl.ds(k * 16, 16)]
                    coef_sub[pl.ds(k * 16, 16)] = ex16 / (dn + 1e-16)
                    return 0
                lax.fori_loop(0, SUB // 16, stepC, 0)

                gd = [None] * NG
                sd = [None] * NG
                gd[0] = fire_g(0)
                for g in range(NG):
                    if g + 1 < NG:
                        if g >= 1:
                            sd[g - 1].wait()
                        gd[g + 1] = fire_g(g + 1)
                    gd[g].wait()
                    scale(g)
                    sd[g] = fire_s(g)
                if NG >= 2:
                    sd[NG - 2].wait()
                sd[NG - 1].wait()
                return 0
            lax.fori_loop(0, ept_b // SUB, subB, 0)

        if edge_split:
            ept_b = EP // 32
            base_b = c * (EP // 2) + s * ept_b
            phaseB(h_tabs[0], base_b, ept_b)
        else:
            ept_b = EP // 16
            base_b = s * ept_b

            @pl.when(c == 0)
            def _():
                phaseB(h_tabs[0], base_b, ept_b)

            @pl.when(c == 1)
            def _():
                phaseB(h_tabs[1], base_b, ept_b)

        plsc.subcore_barrier()
        for cc in range(2):
            @pl.when(c == cc)
            def _(cc=cc):
                pltpu.sync_copy(
                    shared_out.at[pl.ds(soff, STRIPE)],
                    out_h.at[cc, pl.ds(soff, STRIPE)])

    scratch = [
        pltpu.VMEM((NG, G), jnp.int32),
        pltpu.VMEM((NG, G), jnp.int32),
        pltpu.VMEM((SUB,), jnp.float32),
        pltpu.VMEM((SUB,), jnp.float32),
        pltpu.VMEM((SUB,), jnp.float32),
        pltpu.VMEM((SUB,), jnp.float32),
        pltpu.VMEM((G, dout_b), jnp.float32),
        pltpu.VMEM((G, dout_b), jnp.float32),
        pltpu.SemaphoreType.DMA,
        pltpu.SemaphoreType.DMA,
        pltpu.SemaphoreType.DMA,
        pltpu.SemaphoreType.DMA,
        pltpu.SemaphoreType.DMA,
        pltpu.SemaphoreType.DMA,
        pltpu.SemaphoreType.DMA,
        pltpu.VMEM_SHARED((NP,), jnp.float32),
        pltpu.VMEM_SHARED((NP,), jnp.float32),
        pltpu.VMEM_SHARED((NP,), jnp.float32),
        pltpu.VMEM_SHARED((NP, dout_b), jnp.float32),
    ]

    out_type = (jax.ShapeDtypeStruct((2, NP, dout_b), jnp.float32),
                jax.ShapeDtypeStruct((NP,), jnp.float32),
                jax.ShapeDtypeStruct((2, EP), jnp.float32))

    return pl.kernel(body, out_type=out_type, mesh=mesh,
                     scratch_types=scratch,
                     compiler_params=pltpu.CompilerParams(
                         needs_layout_passes=False,
                         use_tc_tiling_on_sc=False))


def _sc_run(dout, edge_split, srcp, dstp, ssrc, sdst, exself, htabs):
    dout_b = dout if edge_split else dout // 2
    G = 128 if dout_b <= 64 else 64
    zeros = jnp.zeros((STRIPE, dout_b), jnp.float32)
    k = _make_sc_layer(dout_b, edge_split, G)
    agg, den, _ex = k(srcp.reshape(EP // G, G), dstp.reshape(EP // G, G),
                      ssrc, sdst, exself, zeros, *htabs)
    return agg, den


def kernel(x, edge_index, batch,
           W1, att_src1, att_dst1, b1,
           W2, att_src2, att_dst2, b2,
           W3, att_src3, att_dst3, b3,
           W4, att_src4, att_dst4, b4,
           lin1_W, lin1_b, lin2_W, lin2_b):
    srcp = jnp.concatenate(
        [edge_index[0], jnp.zeros((EP - E,), jnp.int32)])
    dstp = jnp.concatenate(
        [edge_index[1], jnp.zeros((EP - E,), jnp.int32)])
    xp = jnp.pad(x, ((0, NP - N), (0, 0)))
    batchp = jnp.pad(batch, (0, NP - N), constant_values=BATCHES)

    h1, ss1, sd1, exs1 = _tc_pre1(xp, W1, att_src1, att_dst1, 32)
    agg1, den1 = _sc_run(32, True, srcp, dstp, ss1, sd1, exs1, (h1,))

    h2, ss2, sd2, exs2 = _tc_pre_sum(
        agg1[0], agg1[1], h1, exs1, den1, b1, W2, att_src2, att_dst2, 32, 64)
    agg2, den2 = _sc_run(64, True, srcp, dstp, ss2, sd2, exs2, (h2,))

    h3, ss3, sd3, exs3 = _tc_pre_sum(
        agg2[0], agg2[1], h2, exs2, den2, b2, W3, att_src3, att_dst3, 64, 128)
    agg3, den3 = _sc_run(128, True, srcp, dstp, ss3, sd3, exs3, (h3,))

    h4a, h4b, ss4, sd4, exs4 = _tc_pre4(
        agg3[0], agg3[1], h3, exs3, den3, b3, W4, att_src4, att_dst4)
    agg4, den4 = _sc_run(256, False, srcp, dstp, ss4, sd4, exs4,
                         (h4a, h4b))

    return _tc_head(agg4[0], agg4[1], h4a, h4b, exs4, den4, b4, batchp,
                    lin1_W, lin1_b, lin2_W, lin2_b)


# speedup vs baseline: 26.1265x; 1.1236x over previous
import functools

import jax
import jax.numpy as jnp
from jax import lax
from jax.experimental import pallas as pl
from jax.experimental.pallas import tpu as pltpu
from jax.experimental.pallas import tpu_sc as plsc

N = 10000
NP = 10240
E = 320000
EP = 327680
BATCHES = 128
SUB = 2048
GRP = 128
RB = 1024
STRIPE = NP // 16


def _elu(v):
    return jnp.where(v > 0, v, jnp.exp(jnp.minimum(v, 0.0)) - 1.0)


def _scores(h, asrc_ref, adst_ref):
    ss = jnp.sum(h * asrc_ref[...], axis=1)
    sd = jnp.sum(h * adst_ref[...], axis=1)
    al = ss + sd
    exself = jnp.exp(jnp.where(al >= 0, al, 0.2 * al))
    return ss, sd, exself


def _pre1_body(x_ref, W_ref, asrc_ref, adst_ref,
               h_ref, ssrc_ref, sdst_ref, exself_ref):
    h = jnp.dot(x_ref[...], W_ref[...], preferred_element_type=jnp.float32)
    h_ref[...] = h
    ss, sd, exself = _scores(h, asrc_ref, adst_ref)
    ssrc_ref[...] = ss
    sdst_ref[...] = sd
    exself_ref[...] = exself


def _pre_sum_body(a0_ref, a1_ref, hprev_ref, exs_ref, den0_ref, den1_ref,
                  b_ref, W_ref, asrc_ref, adst_ref,
                  h_ref, ssrc_ref, sdst_ref, exself_ref):
    exs = exs_ref[...].reshape(-1, 1)
    dent = (den0_ref[...] + den1_ref[...] + exs_ref[...] + 1e-16
            ).reshape(-1, 1)
    num = a0_ref[...] + a1_ref[...] + hprev_ref[...] * exs
    x = _elu(num / dent + b_ref[...])
    h = jnp.dot(x, W_ref[...], preferred_element_type=jnp.float32)
    h_ref[...] = h
    ss, sd, exself = _scores(h, asrc_ref, adst_ref)
    ssrc_ref[...] = ss
    sdst_ref[...] = sd
    exself_ref[...] = exself


def _pre4_body(a0_ref, a1_ref, hprev_ref, exs_ref, den_ref, b_ref,
               W_ref, asrc_ref, adst_ref,
               h0_ref, h1_ref, ssrc_ref, sdst_ref, exself_ref):
    exs = exs_ref[...].reshape(-1, 1)
    dent = (den_ref[...] + exs_ref[...] + 1e-16).reshape(-1, 1)
    agg = jnp.concatenate([a0_ref[...], a1_ref[...]], axis=1)
    x = _elu((agg + hprev_ref[...] * exs) / dent + b_ref[...])
    h = jnp.dot(x, W_ref[...], preferred_element_type=jnp.float32)
    h0_ref[...] = h[:, :128]
    h1_ref[...] = h[:, 128:]
    ss, sd, exself = _scores(h, asrc_ref, adst_ref)
    ssrc_ref[...] = ss
    sdst_ref[...] = sd
    exself_ref[...] = exself


def _head_body(a0_ref, a1_ref, h0_ref, h1_ref, exs_ref, den_ref, b_ref,
               batch_ref, l1W_ref, l1b_ref, l2W_ref, l2b_ref,
               out_ref, sums_ref, counts_ref):
    i = pl.program_id(0)

    @pl.when(i == 0)
    def _():
        sums_ref[...] = jnp.zeros_like(sums_ref)
        counts_ref[...] = jnp.zeros_like(counts_ref)

    exs = exs_ref[...].reshape(-1, 1)
    dent = (den_ref[...] + exs_ref[...] + 1e-16).reshape(-1, 1)
    agg = jnp.concatenate([a0_ref[...], a1_ref[...]], axis=1)
    hprev = jnp.concatenate([h0_ref[...], h1_ref[...]], axis=1)
    x = _elu((agg + hprev * exs) / dent + b_ref[...])
    bb = batch_ref[...].reshape(RB, 1)
    P = (bb == lax.broadcasted_iota(jnp.int32, (RB, BATCHES), 1)
         ).astype(jnp.float32)
    sums_ref[...] += lax.dot_general(
        P, x, (((0,), (0,)), ((), ())), preferred_element_type=jnp.float32)
    counts_ref[...] += jnp.sum(P, axis=0).reshape(1, BATCHES)

    @pl.when(i == pl.num_programs(0) - 1)
    def _():
        counts = jnp.maximum(counts_ref[...].reshape(BATCHES, 1), 1.0)
        pooled = sums_ref[...] / counts
        hh = _elu(jnp.dot(pooled, l1W_ref[...],
                          preferred_element_type=jnp.float32) + l1b_ref[...])
        out_ref[...] = jnp.dot(hh, l2W_ref[...],
                               preferred_element_type=jnp.float32) + l2b_ref[...]


def _row_spec(width):
    return pl.BlockSpec((RB, width), lambda i: (i, 0))


def _vec_spec():
    return pl.BlockSpec((RB,), lambda i: (i,))


def _full_spec(shape):
    return pl.BlockSpec(shape, lambda i: tuple(0 for _ in shape))


def _tc_pre1(xp, W, asrc, adst, dout):
    return pl.pallas_call(
        _pre1_body,
        grid=(NP // RB,),
        in_specs=[_row_spec(128), _full_spec(W.shape),
                  _full_spec((1, dout)), _full_spec((1, dout))],
        out_specs=[_row_spec(dout), _vec_spec(), _vec_spec(), _vec_spec()],
        out_shape=[jax.ShapeDtypeStruct((NP, dout), jnp.float32),
                   jax.ShapeDtypeStruct((NP,), jnp.float32),
                   jax.ShapeDtypeStruct((NP,), jnp.float32),
                   jax.ShapeDtypeStruct((NP,), jnp.float32)],
    )(xp, W, asrc.reshape(1, -1), adst.reshape(1, -1))


def _tc_pre_sum(a0, a1, hprev, exs, den0, den1, b, W, asrc, adst,
                din, dout):
    return pl.pallas_call(
        _pre_sum_body,
        grid=(NP // RB,),
        in_specs=[_row_spec(din), _row_spec(din), _row_spec(din),
                  _vec_spec(), _vec_spec(), _vec_spec(),
                  _full_spec((1, din)), _full_spec(W.shape),
                  _full_spec((1, dout)), _full_spec((1, dout))],
        out_specs=[_row_spec(dout), _vec_spec(), _vec_spec(), _vec_spec()],
        out_shape=[jax.ShapeDtypeStruct((NP, dout), jnp.float32),
                   jax.ShapeDtypeStruct((NP,), jnp.float32),
                   jax.ShapeDtypeStruct((NP,), jnp.float32),
                   jax.ShapeDtypeStruct((NP,), jnp.float32)],
    )(a0, a1, hprev, exs, den0, den1, b.reshape(1, -1), W,
      asrc.reshape(1, -1), adst.reshape(1, -1))


def _tc_pre4(a0, a1, hprev, exs, den, b, W, asrc, adst):
    return pl.pallas_call(
        _pre4_body,
        grid=(NP // RB,),
        in_specs=[_row_spec(64), _row_spec(64), _row_spec(128),
                  _vec_spec(), _vec_spec(), _full_spec((1, 128)),
                  _full_spec(W.shape),
                  _full_spec((1, 256)), _full_spec((1, 256))],
        out_specs=[_row_spec(128), _row_spec(128),
                   _vec_spec(), _vec_spec(), _vec_spec()],
        out_shape=[jax.ShapeDtypeStruct((NP, 128), jnp.float32),
                   jax.ShapeDtypeStruct((NP, 128), jnp.float32),
                   jax.ShapeDtypeStruct((NP,), jnp.float32),
                   jax.ShapeDtypeStruct((NP,), jnp.float32),
                   jax.ShapeDtypeStruct((NP,), jnp.float32)],
    )(a0, a1, hprev, exs, den, b.reshape(1, -1), W,
      asrc.reshape(1, -1), adst.reshape(1, -1))


def _tc_head(a0, a1, h0, h1, exs, den, b4, batchp, l1W, l1b, l2W, l2b):
    return pl.pallas_call(
        _head_body,
        grid=(NP // RB,),
        in_specs=[_row_spec(128), _row_spec(128), _row_spec(128),
                  _row_spec(128), _vec_spec(), _vec_spec(),
                  _full_spec((1, 256)), _vec_spec(),
                  _full_spec((256, 128)), _full_spec((1, 128)),
                  _full_spec((128, 10)), _full_spec((1, 10))],
        out_specs=pl.BlockSpec((BATCHES, 10), lambda i: (0, 0)),
        out_shape=jax.ShapeDtypeStruct((BATCHES, 10), jnp.float32),
        scratch_shapes=[pltpu.VMEM((BATCHES, 256), jnp.float32),
                        pltpu.VMEM((1, BATCHES), jnp.float32)],
    )(a0, a1, h0, h1, exs, den, b4.reshape(1, -1), batchp,
      l1W, l1b.reshape(1, -1), l2W, l2b.reshape(1, -1))


def _make_sc_layer(dout_b, edge_split, G, local_tables):
    NG = SUB // G
    KR = G // 16
    mesh = plsc.VectorSubcoreMesh(core_axis_name="c", subcore_axis_name="s",
                                  num_cores=2, num_subcores=16)
    n_h = 1 if edge_split else 2
    NR = NP // 128

    def body(srcp2_h, dstp2_h, ssrc_h, sdst_h, zeros_h, *rest):
        h_tabs = rest[:n_h]
        out_h, den_out_h = rest[n_h], rest[n_h + 1]
        if local_tables:
            (src2_sub, dst2_sub, coef_sub, s_tab, den_v, riota,
             rows0, rows1, sem_l, sem_a, sem_b, gsem0, gsem1, ssem0, ssem1,
             shared_den, shared_out) = rest[n_h + 2:]
        else:
            (src2_sub, dst2_sub, coef_sub, a_sub, b_sub, den_v, riota,
             rows0, rows1, sem_l, sem_a, sem_b, gsem0, gsem1, ssem0, ssem1,
             shared_ssrc, shared_sdst, shared_den, shared_out) = \
                rest[n_h + 2:]
        rows = (rows0, rows1)
        gsem = (gsem0, gsem1)
        ssem = (ssem0, ssem1)

        c = lax.axis_index("c")
        s = lax.axis_index("s")
        soff = pl.multiple_of(s * STRIPE, STRIPE)

        def zden(i, _):
            den_v[i // 8, pl.ds((i % 8) * 16, 16)] = (
                jnp.zeros((16,), jnp.float32))
            return 0
        lax.fori_loop(0, NP // 16, zden, 0)
        for i in range(NR // 16):
            riota[pl.ds(i * 16, 16)] = i * 16 + lax.iota(jnp.int32, 16)

        if local_tables:
            pltpu.sync_copy(ssrc_h, s_tab.at[pl.ds(0, NP)])
            pltpu.sync_copy(sdst_h, s_tab.at[pl.ds(NP, NP)])
        else:
            @pl.when(s == 1)
            def _():
                pltpu.sync_copy(ssrc_h, shared_ssrc)

            @pl.when(s == 2)
            def _():
                pltpu.sync_copy(sdst_h, shared_sdst)

        @pl.when(s == 0)
        def _():
            pltpu.sync_copy(den_v, shared_den)

        pltpu.sync_copy(zeros_h, shared_out.at[pl.ds(soff, STRIPE)])
        plsc.subcore_barrier()

        def load_blocks(off):
            offr = pl.multiple_of(off // G, NG)
            d1 = pltpu.async_copy(srcp2_h.at[pl.ds(offr, NG)], src2_sub,
                                  sem_l)
            d2 = pltpu.async_copy(dstp2_h.at[pl.ds(offr, NG)], dst2_sub,
                                  sem_l)
            return d1, d2

        def run(h_tab, base, ept):
            def fire_g(g):
                return pltpu.async_copy(h_tab.at[src2_sub.at[g]],
                                        rows[g % 2], gsem[g % 2])

            def fire_s(g):
                return pltpu.async_copy(rows[g % 2],
                                        shared_out.at[dst2_sub.at[g]],
                                        ssem[g % 2], add=True)

            def scale(g):
                def stepS(e, _):
                    cfb = plsc.load_gather(
                        coef_sub, [jnp.full((16,), g * G + e, jnp.int32)])
                    r = rows[g % 2]
                    for cb in range(dout_b // 16):
                        v = r[e, pl.ds(cb * 16, 16)]
                        r[e, pl.ds(cb * 16, 16)] = v * cfb
                    return 0
                lax.fori_loop(0, G, stepS, 0)

            def sub_once(sub, _):
                off = pl.multiple_of(base + sub * SUB, SUB)
                d1, d2 = load_blocks(off)
                d1.wait()
                d2.wait()
                gd = [None] * NG
                sd = [None] * NG
                gd[0] = fire_g(0)

                if local_tables:
                    def stepE(k, _):
                        i16 = src2_sub[k // KR, pl.ds((k % KR) * 16, 16)]
                        d16 = dst2_sub[k // KR, pl.ds((k % KR) * 16, 16)]
                        av = plsc.load_gather(s_tab, [i16])
                        bv = plsc.load_gather(s_tab, [d16 + NP])
                        al = av + bv
                        al = jnp.where(al >= 0, al, 0.2 * al)
                        ex = jnp.exp(al)
                        eidx = off + k * 16 + lax.iota(jnp.int32, 16)
                        ex = jnp.where(eidx < E, ex, 0.0)
                        coef_sub[pl.ds(k * 16, 16)] = ex
                        plsc.addupdate_scatter(
                            den_v, [d16 // 128, d16 % 128], ex)
                        return 0
                    lax.fori_loop(0, SUB // 16, stepE, 0)
                else:
                    descs = []
                    for g in range(NG):
                        descs.append(pltpu.async_copy(
                            shared_ssrc.at[src2_sub.at[g]],
                            a_sub.at[pl.ds(g * G, G)], sem_a))
                        descs.append(pltpu.async_copy(
                            shared_sdst.at[dst2_sub.at[g]],
                            b_sub.at[pl.ds(g * G, G)], sem_b))
                    for d in descs:
                        d.wait()

                    def stepE(k, _):
                        d16 = dst2_sub[k // KR, pl.ds((k % KR) * 16, 16)]
                        al = (a_sub[pl.ds(k * 16, 16)]
                              + b_sub[pl.ds(k * 16, 16)])
                        al = jnp.where(al >= 0, al, 0.2 * al)
                        ex = jnp.exp(al)
                        eidx = off + k * 16 + lax.iota(jnp.int32, 16)
                        ex = jnp.where(eidx < E, ex, 0.0)
                        coef_sub[pl.ds(k * 16, 16)] = ex
                        plsc.addupdate_scatter(
                            den_v, [d16 // 128, d16 % 128], ex)
                        return 0
                    lax.fori_loop(0, SUB // 16, stepE, 0)

                for g in range(NG):
                    if g + 1 < NG:
                        if g >= 1:
                            sd[g - 1].wait()
                        gd[g + 1] = fire_g(g + 1)
                    gd[g].wait()
                    scale(g)
                    sd[g] = fire_s(g)
                if NG >= 2:
                    sd[NG - 2].wait()
                sd[NG - 1].wait()
                return 0
            lax.fori_loop(0, ept // SUB, sub_once, 0)

        if edge_split:
            ept = EP // 32
            run(h_tabs[0], c * (EP // 2) + s * ept, ept)
        else:
            ept = EP // 16
            base = s * ept

            @pl.when(c == 0)
            def _():
                run(h_tabs[0], base, ept)

            @pl.when(c == 1)
            def _():
                run(h_tabs[1], base, ept)

        pltpu.sync_copy(den_v, shared_den.at[riota], add=True)
        plsc.subcore_barrier()
        for cc in range(2):
            @pl.when(jnp.logical_and(c == cc, s < 10))
            def _(cc=cc):
                r8 = pl.multiple_of(s * 8, 8)
                pltpu.sync_copy(shared_den.at[pl.ds(r8, 8)],
                                den_out_h.at[cc, pl.ds(r8, 8)])

            @pl.when(c == cc)
            def _(cc=cc):
                pltpu.sync_copy(
                    shared_out.at[pl.ds(soff, STRIPE)],
                    out_h.at[cc, pl.ds(soff, STRIPE)])

    scratch = [
        pltpu.VMEM((NG, G), jnp.int32),
        pltpu.VMEM((NG, G), jnp.int32),
        pltpu.VMEM((SUB,), jnp.float32),
    ]
    if local_tables:
        scratch += [pltpu.VMEM((2 * NP,), jnp.float32)]
    else:
        scratch += [pltpu.VMEM((SUB,), jnp.float32),
                    pltpu.VMEM((SUB,), jnp.float32)]
    scratch += [
        pltpu.VMEM((NR, 128), jnp.float32),
        pltpu.VMEM((NR,), jnp.int32),
        pltpu.VMEM((G, dout_b), jnp.float32),
        pltpu.VMEM((G, dout_b), jnp.float32),
        pltpu.SemaphoreType.DMA,
        pltpu.SemaphoreType.DMA,
        pltpu.SemaphoreType.DMA,
        pltpu.SemaphoreType.DMA,
        pltpu.SemaphoreType.DMA,
        pltpu.SemaphoreType.DMA,
        pltpu.SemaphoreType.DMA,
    ]
    if not local_tables:
        scratch += [pltpu.VMEM_SHARED((NP,), jnp.float32),
                    pltpu.VMEM_SHARED((NP,), jnp.float32)]
    scratch += [
        pltpu.VMEM_SHARED((NR, 128), jnp.float32),
        pltpu.VMEM_SHARED((NP, dout_b), jnp.float32),
    ]

    out_type = (jax.ShapeDtypeStruct((2, NP, dout_b), jnp.float32),
                jax.ShapeDtypeStruct((2, NR, 128), jnp.float32))

    return pl.kernel(body, out_type=out_type, mesh=mesh,
                     scratch_types=scratch,
                     compiler_params=pltpu.CompilerParams(
                         needs_layout_passes=False,
                         use_tc_tiling_on_sc=False))


def _sc_run(dout, edge_split, srcp, dstp, ssrc, sdst, htabs):
    dout_b = dout if edge_split else dout // 2
    G = 128 if dout_b <= 64 else 64
    local_tables = dout_b <= 64
    zeros = jnp.zeros((STRIPE, dout_b), jnp.float32)
    k = _make_sc_layer(dout_b, edge_split, G, local_tables)
    agg, den = k(srcp.reshape(EP // G, G), dstp.reshape(EP // G, G),
                 ssrc, sdst, zeros, *htabs)
    return agg, den.reshape(2, NP)


def kernel(x, edge_index, batch,
           W1, att_src1, att_dst1, b1,
           W2, att_src2, att_dst2, b2,
           W3, att_src3, att_dst3, b3,
           W4, att_src4, att_dst4, b4,
           lin1_W, lin1_b, lin2_W, lin2_b):
    srcp = jnp.concatenate(
        [edge_index[0], jnp.zeros((EP - E,), jnp.int32)])
    dstp = jnp.concatenate(
        [edge_index[1], jnp.zeros((EP - E,), jnp.int32)])
    xp = jnp.pad(x, ((0, NP - N), (0, 0)))
    batchp = jnp.pad(batch, (0, NP - N), constant_values=BATCHES)

    h1, ss1, sd1, exs1 = _tc_pre1(xp, W1, att_src1, att_dst1, 32)
    agg1, den1 = _sc_run(32, True, srcp, dstp, ss1, sd1, (h1,))

    h2, ss2, sd2, exs2 = _tc_pre_sum(
        agg1[0], agg1[1], h1, exs1, den1[0], den1[1], b1,
        W2, att_src2, att_dst2, 32, 64)
    agg2, den2 = _sc_run(64, True, srcp, dstp, ss2, sd2, (h2,))

    h3, ss3, sd3, exs3 = _tc_pre_sum(
        agg2[0], agg2[1], h2, exs2, den2[0], den2[1], b2,
        W3, att_src3, att_dst3, 64, 128)
    agg3, den3 = _sc_run(128, False, srcp, dstp, ss3, sd3,
                         (h3[:, :64], h3[:, 64:]))

    h4a, h4b, ss4, sd4, exs4 = _tc_pre4(
        agg3[0], agg3[1], h3, exs3, den3[0], b3, W4, att_src4, att_dst4)
    agg4, den4 = _sc_run(256, False, srcp, dstp, ss4, sd4, (h4a, h4b))

    return _tc_head(agg4[0], agg4[1], h4a, h4b, exs4, den4[0], b4, batchp,
                    lin1_W, lin1_b, lin2_W, lin2_b)
